# baseline (speedup 1.0000x reference)
import functools

import jax
import jax.numpy as jnp
from jax import lax
from jax.experimental import pallas as pl
from jax.experimental.pallas import tpu as pltpu
from jax.experimental.pallas import tpu_sc as plsc

N = 5000
NP = 5120
CHUNK = 128
KSLOT = 16
C = NP // CHUNK
RB = 512
CAP = 4 * N
KEEP = N

NTILE = 16
TPR = NP // NTILE
GROUP = 32
NGROUP = TPR // GROUP
CANDCAP = 4096
NVREG = CANDCAP // 16
SLOTS = C * KSLOT
SVREG = SLOTS // 16


def _passa_body(xmin_r, ymin_r, xmax_r, ymax_r,
                xmin_c, ymin_c, xmax_c, ymax_c, jtab_ref, scr):
    ri = pl.program_id(0)
    a_xmin = xmin_r[...]
    a_ymin = ymin_r[...]
    a_xmax = xmax_r[...]
    a_ymax = ymax_r[...]

    row_id = ri * RB + lax.broadcasted_iota(jnp.int32, (CHUNK, RB), 1)

    ii = lax.broadcasted_iota(jnp.int32, (CHUNK, CHUNK), 0)
    jj = lax.broadcasted_iota(jnp.int32, (CHUNK, CHUNK), 1)
    ltl = (jj <= ii).astype(jnp.float32)
    ones_row = jnp.ones((1, CHUNK), jnp.float32)
    jloc_row = lax.broadcasted_iota(jnp.int32, (1, CHUNK), 1).astype(jnp.float32)
    red = jnp.concatenate([ones_row, jloc_row], axis=0)

    for c in range(C):
        live = ri * (RB // CHUNK) <= c

        @pl.when(live)
        def _compute(c=c):
            sl = slice(c * CHUNK, (c + 1) * CHUNK)
            b_xmin = xmin_c[sl, :]
            b_ymin = ymin_c[sl, :]
            b_xmax = xmax_c[sl, :]
            b_ymax = ymax_c[sl, :]
            col_id = c * CHUNK + lax.broadcasted_iota(jnp.int32, (CHUNK, RB), 0)

            hit = ((a_xmin <= b_xmax) & (b_xmin <= a_xmax)
                   & (a_ymin <= b_ymax) & (b_ymin <= a_ymax)
                   & (col_id > row_id))
            cf = hit.astype(jnp.float32)
            cum = jax.lax.dot(ltl, cf, precision=jax.lax.Precision.HIGHEST)
            skey = jnp.where(hit, cum - cf, jnp.float32(-1.0))

            for s in range(KSLOT):
                m = (skey == float(s)).astype(jnp.float32)
                d2 = jax.lax.dot(red, m, precision=jax.lax.Precision.HIGHEST)
                jval = float(c * CHUNK) * d2[0:1, :] + d2[1:2, :]
                scr[c * KSLOT + s, :] = jval[0, :]

        @pl.when(jnp.logical_not(live))
        def _zero(c=c):
            scr[c * KSLOT:(c + 1) * KSLOT, :] = jnp.zeros(
                (KSLOT, RB), jnp.float32)

    jtab_ref[...] = scr[...].T.astype(jnp.int32)


def _passa(xmin, ymin, xmax, ymax):
    grid = (NP // RB,)
    return pl.pallas_call(
        _passa_body,
        grid=grid,
        in_specs=[
            pl.BlockSpec((1, RB), lambda i: (0, i)),
            pl.BlockSpec((1, RB), lambda i: (0, i)),
            pl.BlockSpec((1, RB), lambda i: (0, i)),
            pl.BlockSpec((1, RB), lambda i: (0, i)),
            pl.BlockSpec((NP, 1), lambda i: (0, 0)),
            pl.BlockSpec((NP, 1), lambda i: (0, 0)),
            pl.BlockSpec((NP, 1), lambda i: (0, 0)),
            pl.BlockSpec((NP, 1), lambda i: (0, 0)),
        ],
        out_specs=pl.BlockSpec((RB, SLOTS), lambda i: (i, 0)),
        out_shape=jax.ShapeDtypeStruct((NP, SLOTS), jnp.int32),
        scratch_shapes=[pltpu.VMEM((SLOTS, RB), jnp.float32)],
    )(xmin.reshape(1, NP), ymin.reshape(1, NP),
      xmax.reshape(1, NP), ymax.reshape(1, NP),
      xmin.reshape(NP, 1), ymin.reshape(NP, 1),
      xmax.reshape(NP, 1), ymax.reshape(NP, 1))


def _iota16():
    return lax.iota(jnp.int32, 16)


def _splat(x):
    return jnp.full((16,), x, jnp.int32)


def _sc_body(jtab, xmin_h, ymin_h, xmax_h, ymax_h, scores_h, zeros_h,
             out_h,
             jbuf, xmin, ymin, xmax, ymax, scores,
             cand_i, cand_j, cand_d, delta, acc, stage, rdbk,
             sh_cnt, sh_delta):
    cid = lax.axis_index("c")
    tid = lax.axis_index("s")

    @pl.when(cid == 0)
    def _work():
        it16 = _iota16()
        rbase = tid * TPR

        pltpu.sync_copy(xmin_h, xmin)
        pltpu.sync_copy(ymin_h, ymin)
        pltpu.sync_copy(xmax_h, xmax)
        pltpu.sync_copy(ymax_h, ymax)
        pltpu.sync_copy(scores_h, scores)
        pltpu.sync_copy(zeros_h, delta)

        def _zb(k, _):
            z = jnp.zeros((16,), jnp.int32)
            cand_i[pl.ds(k * 16, 16)] = z
            cand_j[pl.ds(k * 16, 16)] = z
            return 0
        lax.fori_loop(0, NVREG, _zb, 0)

        def _group(g, off):
            pltpu.sync_copy(jtab.at[pl.ds((rbase + g * GROUP) * SLOTS,
                                          GROUP * SLOTS)], jbuf)

            def _row(rb, off):
                row_i = rbase + g * GROUP + rb
                rb_off = _splat(rb * SLOTS)
                for c in range(SVREG):
                    v = plsc.load_gather(jbuf, [rb_off + _splat(c * 16) + it16])
                    m = v > 0
                    mi = m.astype(jnp.int32)
                    pos = _splat(off) + plsc.cumsum(mi) - mi
                    plsc.store_scatter(cand_j, [pos], v, mask=m)
                    plsc.store_scatter(cand_i, [pos], _splat(row_i), mask=m)
                    off = off + jnp.sum(mi)
                return jnp.minimum(off, CANDCAP - 16)
            return lax.fori_loop(0, GROUP, _row, off)

        t_cnt = lax.fori_loop(0, NGROUP, _group, jnp.int32(0))

        def _publish(val):
            stage[...] = _splat(val)
            pltpu.sync_copy(stage, sh_cnt.at[pl.ds(tid * 16, 16)])
            plsc.subcore_barrier()
            pltpu.sync_copy(sh_cnt, rdbk)

            def _acc(u, bt):
                base, tot = bt
                cu = jnp.max(plsc.load_gather(rdbk, [_splat(u) * _splat(16) + it16]))
                base = base + jnp.where(u < tid, cu, 0)
                return (base, tot + cu)
            base, tot = lax.fori_loop(0, NTILE, _acc,
                                      (jnp.int32(0), jnp.int32(0)))
            plsc.subcore_barrier()
            return base, tot

        base_t, _tot = _publish(t_cnt)
        m_t = jnp.clip(CAP - base_t, 0, t_cnt)
        nvd = (t_cnt + 15) // 16

        def _depth(k, _):
            lanes = _splat(k * 16) + it16
            valid = lanes < m_t
            ii_ = cand_i[pl.ds(k * 16, 16)]
            jj_ = cand_j[pl.ds(k * 16, 16)]
            axmin = plsc.load_gather(xmin, [ii_])
            axmax = plsc.load_gather(xmax, [ii_])
            bxmin = plsc.load_gather(xmin, [jj_])
            bxmax = plsc.load_gather(xmax, [jj_])
            aymin = plsc.load_gather(ymin, [ii_])
            aymax = plsc.load_gather(ymax, [ii_])
            bymin = plsc.load_gather(ymin, [jj_])
            bymax = plsc.load_gather(ymax, [jj_])
            ox = jnp.minimum(axmax, bxmax) - jnp.maximum(axmin, bxmin)
            oy = jnp.minimum(aymax, bymax) - jnp.maximum(aymin, bymin)
            d = jnp.minimum(ox, oy)
            d = jnp.where(valid & (d > 0), d, jnp.float32(-1.0))
            cand_d[pl.ds(k * 16, 16)] = d
            return 0
        lax.fori_loop(0, nvd, _depth, 0)

        def _count_ge(thr):
            def _cnt(k, c):
                d = cand_d[pl.ds(k * 16, 16)]
                di = plsc.bitcast(d, jnp.int32)
                return c + jnp.sum((di >= thr).astype(jnp.int32))
            local = lax.fori_loop(0, nvd, _cnt, jnp.int32(0))
            _, tot = _publish(local)
            return tot

        def _bs(_, lh):
            lo, hi = lh
            mid = (lo + hi) // 2
            c = _count_ge(mid)
            take = c >= KEEP
            return (jnp.where(take, mid, lo), jnp.where(take, hi, mid))
        lo, _hi = lax.fori_loop(0, 31, _bs,
                                (jnp.int32(1), jnp.int32(0x40C00002)))

        n_gt = _count_ge(lo + 1)
        extra = KEEP - n_gt

        def _ecnt(k, c):
            d = cand_d[pl.ds(k * 16, 16)]
            di = plsc.bitcast(d, jnp.int32)
            return c + jnp.sum((di == lo).astype(jnp.int32))
        e_t = lax.fori_loop(0, nvd, _ecnt, jnp.int32(0))
        base_e, _te = _publish(e_t)
        k_t = jnp.clip(extra - base_e, 0, e_t)

        def _sel(k, eqrun):
            d = cand_d[pl.ds(k * 16, 16)]
            di = plsc.bitcast(d, jnp.int32)
            sel_gt = di >= (lo + 1)
            meq = di == lo
            mi = meq.astype(jnp.int32)
            eqpos = _splat(eqrun) + plsc.cumsum(mi) - mi
            sel = sel_gt | (meq & (eqpos < k_t))
            ii_ = cand_i[pl.ds(k * 16, 16)]
            jj_ = cand_j[pl.ds(k * 16, 16)]
            axmin = plsc.load_gather(xmin, [ii_])
            axmax = plsc.load_gather(xmax, [ii_])
            bxmin = plsc.load_gather(xmin, [jj_])
            bxmax = plsc.load_gather(xmax, [jj_])
            aymin = plsc.load_gather(ymin, [ii_])
            aymax = plsc.load_gather(ymax, [ii_])
            bymin = plsc.load_gather(ymin, [jj_])
            bymax = plsc.load_gather(ymax, [jj_])
            ox = jnp.minimum(axmax, bxmax) - jnp.maximum(axmin, bxmin)
            oy = jnp.minimum(aymax, bymax) - jnp.maximum(aymin, bymin)
            cxa = (axmin + axmax) * 0.5
            cya = (aymin + aymax) * 0.5
            cxb = (bxmin + bxmax) * 0.5
            cyb = (bymin + bymax) * 0.5
            one = jnp.full((16,), 1.0, jnp.float32)
            sx = jnp.where(cxb >= cxa, one, -one)
            sy = jnp.where(cyb >= cya, one, -one)
            use_x = ox < oy
            zero = jnp.zeros((16,), jnp.float32)
            px = jnp.where(use_x, sx * ox, zero)
            py = jnp.where(use_x, zero, sy * oy)
            wi = plsc.load_gather(scores, [ii_])
            wj = plsc.load_gather(scores, [jj_])
            wsum = wi + wj
            mf = jnp.where(sel, one, zero)
            fi = wj / wsum * mf
            fj = wi / wsum * mf
            dix = -px * fi
            diy = -py * fi
            djx = px * fj
            djy = py * fj
            four = _splat(4)
            ibase = ii_ * four
            jbase = jj_ * four
            for col, val in ((0, dix), (1, diy), (2, dix), (3, diy)):
                plsc.addupdate_scatter(delta, [ibase + _splat(col)], val)
            for col, val in ((0, djx), (1, djy), (2, djx), (3, djy)):
                plsc.addupdate_scatter(delta, [jbase + _splat(col)], val)
            return eqrun + jnp.sum(mi)
        lax.fori_loop(0, nvd, _sel, jnp.int32(0))

        pltpu.sync_copy(delta, sh_delta.at[pl.ds(tid * NP * 4, NP * 4)])
        plsc.subcore_barrier()
        pltpu.sync_copy(zeros_h.at[pl.ds(0, TPR * 4)], acc)

        def _red(u, _):
            pltpu.sync_copy(sh_delta.at[pl.ds(u * NP * 4 + rbase * 4, TPR * 4)],
                            delta.at[pl.ds(0, TPR * 4)])

            def _addv(k, _):
                cur = acc[pl.ds(k * 16, 16)]
                add = delta[pl.ds(k * 16, 16)]
                acc[pl.ds(k * 16, 16)] = cur + add
                return 0
            lax.fori_loop(0, TPR * 4 // 16, _addv, 0)
            return 0
        lax.fori_loop(0, NTILE, _red, 0)

        pltpu.sync_copy(acc, out_h.at[pl.ds(rbase * 4, TPR * 4)])
        plsc.subcore_barrier()


def _passb(jtab, xmin, ymin, xmax, ymax, scores_p, zeros4):
    mesh = plsc.VectorSubcoreMesh(core_axis_name="c", subcore_axis_name="s")
    f32 = jnp.float32
    kern = functools.partial(
        pl.kernel,
        mesh=mesh,
        compiler_params=pltpu.CompilerParams(needs_layout_passes=False),
        out_type=jax.ShapeDtypeStruct((NP * 4,), f32),
        scratch_types=[
            pltpu.VMEM((GROUP * SLOTS,), jnp.int32),
            pltpu.VMEM((NP,), f32),
            pltpu.VMEM((NP,), f32),
            pltpu.VMEM((NP,), f32),
            pltpu.VMEM((NP,), f32),
            pltpu.VMEM((NP,), f32),
            pltpu.VMEM((CANDCAP,), jnp.int32),
            pltpu.VMEM((CANDCAP,), jnp.int32),
            pltpu.VMEM((CANDCAP,), f32),
            pltpu.VMEM((NP * 4,), f32),
            pltpu.VMEM((TPR * 4,), f32),
            pltpu.VMEM((16,), jnp.int32),
            pltpu.VMEM((NTILE * 16,), jnp.int32),
            pltpu.VMEM_SHARED((NTILE * 16,), jnp.int32),
            pltpu.VMEM_SHARED((NTILE * NP * 4,), f32),
        ],
    )
    return kern(_sc_body)(jtab, xmin, ymin, xmax, ymax, scores_p, zeros4)


def _passc_body(b_ref, d_ref, o_ref):
    o_ref[...] = b_ref[...] + d_ref[...]


def _passc(boxes_p, delta):
    return pl.pallas_call(
        _passc_body,
        out_shape=jax.ShapeDtypeStruct((NP, 4), jnp.float32),
    )(boxes_p, delta)


def _pad_cols(boxes):
    pad = NP - N
    xmin = jnp.pad(boxes[:, 0], (0, pad), constant_values=3.0e30)
    ymin = jnp.pad(boxes[:, 1], (0, pad), constant_values=3.0e30)
    xmax = jnp.pad(boxes[:, 2], (0, pad), constant_values=-3.0e30)
    ymax = jnp.pad(boxes[:, 3], (0, pad), constant_values=-3.0e30)
    return xmin, ymin, xmax, ymax


def kernel(boxes, scores):
    xmin, ymin, xmax, ymax = _pad_cols(boxes)
    jtab = _passa(xmin, ymin, xmax, ymax)
    scores_p = jnp.pad(scores, (0, NP - N))
    zeros4 = jnp.zeros((NP * 4,), jnp.float32)
    delta = _passb(jtab.reshape(NP * SLOTS), xmin, ymin, xmax, ymax,
                   scores_p, zeros4)
    boxes_p = jnp.pad(boxes, ((0, NP - N), (0, 0)))
    out = _passc(boxes_p, delta.reshape(NP, 4))
    return out[:N]


# speedup vs baseline: 2.0283x; 2.0283x over previous
import functools

import jax
import jax.numpy as jnp
from jax import lax
from jax.experimental import pallas as pl
from jax.experimental.pallas import tpu as pltpu
from jax.experimental.pallas import tpu_sc as plsc

N = 5000
NP = 5120
CHUNK = 128
KSLOT = 16
C = NP // CHUNK
RB = 512
CAP = 4 * N
KEEP = N

NTILE = 16
TPR = NP // NTILE
GROUP = 32
NGROUP = TPR // GROUP
CANDCAP = 4096
NVREG = CANDCAP // 16
SLOTS = C * KSLOT
SVREG = SLOTS // 16
CPAD = 48


def _passa_body(xmin_r, ymin_r, xmax_r, ymax_r,
                xmin_c, ymin_c, xmax_c, ymax_c, jtab_ref, cnt_ref, scr, scr2):
    ri = pl.program_id(0)
    a_xmin = xmin_r[...]
    a_ymin = ymin_r[...]
    a_xmax = xmax_r[...]
    a_ymax = ymax_r[...]

    row_id = ri * RB + lax.broadcasted_iota(jnp.int32, (CHUNK, RB), 1)

    ii = lax.broadcasted_iota(jnp.int32, (CHUNK, CHUNK), 0)
    jj = lax.broadcasted_iota(jnp.int32, (CHUNK, CHUNK), 1)
    ltl = (jj <= ii).astype(jnp.float32)
    scr2[C:CPAD, :] = jnp.zeros((CPAD - C, RB), jnp.float32)

    for c in range(C):
        live = ri * (RB // CHUNK) <= c

        @pl.when(live)
        def _compute(c=c):
            sl = slice(c * CHUNK, (c + 1) * CHUNK)
            b_xmin = xmin_c[sl, :]
            b_ymin = ymin_c[sl, :]
            b_xmax = xmax_c[sl, :]
            b_ymax = ymax_c[sl, :]
            col_id = c * CHUNK + lax.broadcasted_iota(jnp.int32, (CHUNK, RB), 0)

            hit = ((a_xmin <= b_xmax) & (b_xmin <= a_xmax)
                   & (a_ymin <= b_ymax) & (b_ymin <= a_ymax)
                   & (col_id > row_id))
            cf = hit.astype(jnp.float32)
            cum = jax.lax.dot(ltl, cf, precision=jax.lax.Precision.HIGHEST)
            skey = jnp.where(hit, cum - cf, jnp.float32(-1.0))
            scr2[c, :] = cum[CHUNK - 1, :]

            colf = col_id.astype(jnp.float32)
            for s in range(KSLOT):
                m = skey == float(s)
                jval = jnp.sum(jnp.where(m, colf, 0.0), axis=0)
                scr[c * KSLOT + s, :] = jval

        @pl.when(jnp.logical_not(live))
        def _zero(c=c):
            scr[c * KSLOT:(c + 1) * KSLOT, :] = jnp.zeros(
                (KSLOT, RB), jnp.float32)
            scr2[c, :] = jnp.zeros((RB,), jnp.float32)

    jtab_ref[...] = scr[...].T.astype(jnp.int32)
    cnt_ref[...] = scr2[...].T.astype(jnp.int32)


def _passa(xmin, ymin, xmax, ymax):
    grid = (NP // RB,)
    return pl.pallas_call(
        _passa_body,
        grid=grid,
        in_specs=[
            pl.BlockSpec((1, RB), lambda i: (0, i)),
            pl.BlockSpec((1, RB), lambda i: (0, i)),
            pl.BlockSpec((1, RB), lambda i: (0, i)),
            pl.BlockSpec((1, RB), lambda i: (0, i)),
            pl.BlockSpec((NP, 1), lambda i: (0, 0)),
            pl.BlockSpec((NP, 1), lambda i: (0, 0)),
            pl.BlockSpec((NP, 1), lambda i: (0, 0)),
            pl.BlockSpec((NP, 1), lambda i: (0, 0)),
        ],
        out_specs=[pl.BlockSpec((RB, SLOTS), lambda i: (i, 0)),
                   pl.BlockSpec((RB, CPAD), lambda i: (i, 0))],
        out_shape=[jax.ShapeDtypeStruct((NP, SLOTS), jnp.int32),
                   jax.ShapeDtypeStruct((NP, CPAD), jnp.int32)],
        scratch_shapes=[pltpu.VMEM((SLOTS, RB), jnp.float32),
                        pltpu.VMEM((CPAD, RB), jnp.float32)],
    )(xmin.reshape(1, NP), ymin.reshape(1, NP),
      xmax.reshape(1, NP), ymax.reshape(1, NP),
      xmin.reshape(NP, 1), ymin.reshape(NP, 1),
      xmax.reshape(NP, 1), ymax.reshape(NP, 1))


def _iota16():
    return lax.iota(jnp.int32, 16)


def _splat(x):
    return jnp.full((16,), x, jnp.int32)


def _sc_body(jtab, cnt, xmin_h, ymin_h, xmax_h, ymax_h, scores_h, zeros_h,
             out_h,
             jbuf, cbuf, clist, xmin, ymin, xmax, ymax, scores,
             cand_i, cand_j, cand_d, delta, acc, stage, rdbk,
             sh_cnt, sh_delta):
    cid = lax.axis_index("c")
    tid = lax.axis_index("s")

    @pl.when(cid == 0)
    def _work():
        it16 = _iota16()
        rbase = tid * TPR

        pltpu.sync_copy(xmin_h, xmin)
        pltpu.sync_copy(ymin_h, ymin)
        pltpu.sync_copy(xmax_h, xmax)
        pltpu.sync_copy(ymax_h, ymax)
        pltpu.sync_copy(scores_h, scores)
        pltpu.sync_copy(zeros_h, delta)

        def _zb(k, _):
            z = jnp.zeros((16,), jnp.int32)
            cand_i[pl.ds(k * 16, 16)] = z
            cand_j[pl.ds(k * 16, 16)] = z
            return 0
        lax.fori_loop(0, NVREG, _zb, 0)

        def _group(g, off):
            pltpu.sync_copy(jtab.at[pl.ds((rbase + g * GROUP) * SLOTS,
                                          GROUP * SLOTS)], jbuf)
            pltpu.sync_copy(cnt.at[pl.ds((rbase + g * GROUP) * CPAD,
                                         GROUP * CPAD)], cbuf)

            def _row(rb, off):
                row_i = rbase + g * GROUP + rb
                cb = _splat(rb * CPAD)
                nc = jnp.int32(0)
                for q in range(CPAD // 16):
                    cq = plsc.load_gather(cbuf, [cb + _splat(q * 16) + it16])
                    mq = cq > 0
                    mqi = mq.astype(jnp.int32)
                    posq = _splat(nc) + plsc.cumsum(mqi) - mqi
                    plsc.store_scatter(clist, [posq], _splat(q * 16) + it16,
                                       mask=mq)
                    nc = nc + jnp.sum(mqi)

                def _chunk(ci, off):
                    c = jnp.max(plsc.load_gather(clist, [_splat(ci)]))
                    v = plsc.load_gather(
                        jbuf, [_splat(rb * SLOTS + c * 16) + it16])
                    m = v > 0
                    mi = m.astype(jnp.int32)
                    pos = _splat(off) + plsc.cumsum(mi) - mi
                    plsc.store_scatter(cand_j, [pos], v, mask=m)
                    plsc.store_scatter(cand_i, [pos], _splat(row_i), mask=m)
                    return jnp.minimum(off + jnp.sum(mi), CANDCAP - 16)
                return lax.fori_loop(0, nc, _chunk, off)
            return lax.fori_loop(0, GROUP, _row, off)

        t_cnt = lax.fori_loop(0, NGROUP, _group, jnp.int32(0))

        def _publish(val):
            stage[...] = _splat(val)
            pltpu.sync_copy(stage, sh_cnt.at[pl.ds(tid * 16, 16)])
            plsc.subcore_barrier()
            pltpu.sync_copy(sh_cnt, rdbk)

            def _acc(u, bt):
                base, tot = bt
                cu = jnp.max(plsc.load_gather(rdbk, [_splat(u) * _splat(16) + it16]))
                base = base + jnp.where(u < tid, cu, 0)
                return (base, tot + cu)
            base, tot = lax.fori_loop(0, NTILE, _acc,
                                      (jnp.int32(0), jnp.int32(0)))
            plsc.subcore_barrier()
            return base, tot

        base_t, _tot = _publish(t_cnt)
        m_t = jnp.clip(CAP - base_t, 0, t_cnt)
        nvd = (t_cnt + 15) // 16

        def _depth(k, _):
            lanes = _splat(k * 16) + it16
            valid = lanes < m_t
            ii_ = cand_i[pl.ds(k * 16, 16)]
            jj_ = cand_j[pl.ds(k * 16, 16)]
            axmin = plsc.load_gather(xmin, [ii_])
            axmax = plsc.load_gather(xmax, [ii_])
            bxmin = plsc.load_gather(xmin, [jj_])
            bxmax = plsc.load_gather(xmax, [jj_])
            aymin = plsc.load_gather(ymin, [ii_])
            aymax = plsc.load_gather(ymax, [ii_])
            bymin = plsc.load_gather(ymin, [jj_])
            bymax = plsc.load_gather(ymax, [jj_])
            ox = jnp.minimum(axmax, bxmax) - jnp.maximum(axmin, bxmin)
            oy = jnp.minimum(aymax, bymax) - jnp.maximum(aymin, bymin)
            d = jnp.minimum(ox, oy)
            d = jnp.where(valid & (d > 0), d, jnp.float32(-1.0))
            cand_d[pl.ds(k * 16, 16)] = d
            return 0
        lax.fori_loop(0, nvd, _depth, 0)

        def _count_ge(thr):
            def _cnt(k, c):
                d = cand_d[pl.ds(k * 16, 16)]
                di = plsc.bitcast(d, jnp.int32)
                return c + jnp.sum((di >= thr).astype(jnp.int32))
            local = lax.fori_loop(0, nvd, _cnt, jnp.int32(0))
            _, tot = _publish(local)
            return tot

        def _bs(_, lh):
            lo, hi = lh
            mid = (lo + hi) // 2
            c = _count_ge(mid)
            take = c >= KEEP
            return (jnp.where(take, mid, lo), jnp.where(take, hi, mid))
        lo, _hi = lax.fori_loop(0, 31, _bs,
                                (jnp.int32(1), jnp.int32(0x40C00002)))

        n_gt = _count_ge(lo + 1)
        extra = KEEP - n_gt

        def _ecnt(k, c):
            d = cand_d[pl.ds(k * 16, 16)]
            di = plsc.bitcast(d, jnp.int32)
            return c + jnp.sum((di == lo).astype(jnp.int32))
        e_t = lax.fori_loop(0, nvd, _ecnt, jnp.int32(0))
        base_e, _te = _publish(e_t)
        k_t = jnp.clip(extra - base_e, 0, e_t)

        def _sel(k, eqrun):
            d = cand_d[pl.ds(k * 16, 16)]
            di = plsc.bitcast(d, jnp.int32)
            sel_gt = di >= (lo + 1)
            meq = di == lo
            mi = meq.astype(jnp.int32)
            eqpos = _splat(eqrun) + plsc.cumsum(mi) - mi
            sel = sel_gt | (meq & (eqpos < k_t))
            ii_ = cand_i[pl.ds(k * 16, 16)]
            jj_ = cand_j[pl.ds(k * 16, 16)]
            axmin = plsc.load_gather(xmin, [ii_])
            axmax = plsc.load_gather(xmax, [ii_])
            bxmin = plsc.load_gather(xmin, [jj_])
            bxmax = plsc.load_gather(xmax, [jj_])
            aymin = plsc.load_gather(ymin, [ii_])
            aymax = plsc.load_gather(ymax, [ii_])
            bymin = plsc.load_gather(ymin, [jj_])
            bymax = plsc.load_gather(ymax, [jj_])
            ox = jnp.minimum(axmax, bxmax) - jnp.maximum(axmin, bxmin)
            oy = jnp.minimum(aymax, bymax) - jnp.maximum(aymin, bymin)
            cxa = (axmin + axmax) * 0.5
            cya = (aymin + aymax) * 0.5
            cxb = (bxmin + bxmax) * 0.5
            cyb = (bymin + bymax) * 0.5
            one = jnp.full((16,), 1.0, jnp.float32)
            sx = jnp.where(cxb >= cxa, one, -one)
            sy = jnp.where(cyb >= cya, one, -one)
            use_x = ox < oy
            zero = jnp.zeros((16,), jnp.float32)
            px = jnp.where(use_x, sx * ox, zero)
            py = jnp.where(use_x, zero, sy * oy)
            wi = plsc.load_gather(scores, [ii_])
            wj = plsc.load_gather(scores, [jj_])
            wsum = wi + wj
            mf = jnp.where(sel, one, zero)
            fi = wj / wsum * mf
            fj = wi / wsum * mf
            dix = -px * fi
            diy = -py * fi
            djx = px * fj
            djy = py * fj
            four = _splat(4)
            ibase = ii_ * four
            jbase = jj_ * four
            for col, val in ((0, dix), (1, diy), (2, dix), (3, diy)):
                plsc.addupdate_scatter(delta, [ibase + _splat(col)], val)
            for col, val in ((0, djx), (1, djy), (2, djx), (3, djy)):
                plsc.addupdate_scatter(delta, [jbase + _splat(col)], val)
            return eqrun + jnp.sum(mi)
        lax.fori_loop(0, nvd, _sel, jnp.int32(0))

        pltpu.sync_copy(delta, sh_delta.at[pl.ds(tid * NP * 4, NP * 4)])
        plsc.subcore_barrier()
        pltpu.sync_copy(zeros_h.at[pl.ds(0, TPR * 4)], acc)

        def _red(u, _):
            pltpu.sync_copy(sh_delta.at[pl.ds(u * NP * 4 + rbase * 4, TPR * 4)],
                            delta.at[pl.ds(0, TPR * 4)])

            def _addv(k, _):
                cur = acc[pl.ds(k * 16, 16)]
                add = delta[pl.ds(k * 16, 16)]
                acc[pl.ds(k * 16, 16)] = cur + add
                return 0
            lax.fori_loop(0, TPR * 4 // 16, _addv, 0)
            return 0
        lax.fori_loop(0, NTILE, _red, 0)

        pltpu.sync_copy(acc, out_h.at[pl.ds(rbase * 4, TPR * 4)])
        plsc.subcore_barrier()


def _passb(jtab, cnt, xmin, ymin, xmax, ymax, scores_p, zeros4):
    mesh = plsc.VectorSubcoreMesh(core_axis_name="c", subcore_axis_name="s")
    f32 = jnp.float32
    kern = functools.partial(
        pl.kernel,
        mesh=mesh,
        compiler_params=pltpu.CompilerParams(needs_layout_passes=False),
        out_type=jax.ShapeDtypeStruct((NP * 4,), f32),
        scratch_types=[
            pltpu.VMEM((GROUP * SLOTS,), jnp.int32),
            pltpu.VMEM((GROUP * CPAD,), jnp.int32),
            pltpu.VMEM((64,), jnp.int32),
            pltpu.VMEM((NP,), f32),
            pltpu.VMEM((NP,), f32),
            pltpu.VMEM((NP,), f32),
            pltpu.VMEM((NP,), f32),
            pltpu.VMEM((NP,), f32),
            pltpu.VMEM((CANDCAP,), jnp.int32),
            pltpu.VMEM((CANDCAP,), jnp.int32),
            pltpu.VMEM((CANDCAP,), f32),
            pltpu.VMEM((NP * 4,), f32),
            pltpu.VMEM((TPR * 4,), f32),
            pltpu.VMEM((16,), jnp.int32),
            pltpu.VMEM((NTILE * 16,), jnp.int32),
            pltpu.VMEM_SHARED((NTILE * 16,), jnp.int32),
            pltpu.VMEM_SHARED((NTILE * NP * 4,), f32),
        ],
    )
    return kern(_sc_body)(jtab, cnt, xmin, ymin, xmax, ymax, scores_p, zeros4)


def _passc_body(b_ref, d_ref, o_ref):
    o_ref[...] = b_ref[...] + d_ref[...]


def _passc(boxes_p, delta):
    return pl.pallas_call(
        _passc_body,
        out_shape=jax.ShapeDtypeStruct((NP, 4), jnp.float32),
    )(boxes_p, delta)


def _pad_cols(boxes):
    pad = NP - N
    xmin = jnp.pad(boxes[:, 0], (0, pad), constant_values=3.0e30)
    ymin = jnp.pad(boxes[:, 1], (0, pad), constant_values=3.0e30)
    xmax = jnp.pad(boxes[:, 2], (0, pad), constant_values=-3.0e30)
    ymax = jnp.pad(boxes[:, 3], (0, pad), constant_values=-3.0e30)
    return xmin, ymin, xmax, ymax


def kernel(boxes, scores):
    xmin, ymin, xmax, ymax = _pad_cols(boxes)
    jtab, cnt = _passa(xmin, ymin, xmax, ymax)
    scores_p = jnp.pad(scores, (0, NP - N))
    zeros4 = jnp.zeros((NP * 4,), jnp.float32)
    delta = _passb(jtab.reshape(NP * SLOTS), cnt.reshape(NP * CPAD),
                   xmin, ymin, xmax, ymax, scores_p, zeros4)
    boxes_p = jnp.pad(boxes, ((0, NP - N), (0, 0)))
    out = _passc(boxes_p, delta.reshape(NP, 4))
    return out[:N]


# speedup vs baseline: 2.0678x; 1.0195x over previous
import functools

import jax
import jax.numpy as jnp
from jax import lax
from jax.experimental import pallas as pl
from jax.experimental.pallas import tpu as pltpu
from jax.experimental.pallas import tpu_sc as plsc

N = 5000
NP = 5120
CHUNK = 128
KSLOT = 16
C = NP // CHUNK
RB = 512
CAP = 4 * N
KEEP = N

NTILE = 16
TPR = NP // NTILE
GROUP = 32
NGROUP = TPR // GROUP
CANDCAP = 4096
NVREG = CANDCAP // 16
SLOTS = C * KSLOT
SVREG = SLOTS // 16
CPAD = 48


def _passa_body(xmin_r, ymin_r, xmax_r, ymax_r,
                xmin_c, ymin_c, xmax_c, ymax_c, jtab_ref, cnt_ref, scr, scr2):
    ri = pl.program_id(0)
    a_xmin = xmin_r[...]
    a_ymin = ymin_r[...]
    a_xmax = xmax_r[...]
    a_ymax = ymax_r[...]

    row_id = ri * RB + lax.broadcasted_iota(jnp.int32, (CHUNK, RB), 1)

    ii = lax.broadcasted_iota(jnp.int32, (CHUNK, CHUNK), 0)
    jj = lax.broadcasted_iota(jnp.int32, (CHUNK, CHUNK), 1)
    ltl = (jj <= ii).astype(jnp.float32)
    scr2[C:CPAD, :] = jnp.zeros((CPAD - C, RB), jnp.float32)

    for c in range(C):
        live = ri * (RB // CHUNK) <= c

        @pl.when(live)
        def _compute(c=c):
            sl = slice(c * CHUNK, (c + 1) * CHUNK)
            b_xmin = xmin_c[sl, :]
            b_ymin = ymin_c[sl, :]
            b_xmax = xmax_c[sl, :]
            b_ymax = ymax_c[sl, :]
            col_id = c * CHUNK + lax.broadcasted_iota(jnp.int32, (CHUNK, RB), 0)

            hit = ((a_xmin <= b_xmax) & (b_xmin <= a_xmax)
                   & (a_ymin <= b_ymax) & (b_ymin <= a_ymax)
                   & (col_id > row_id))
            cf = hit.astype(jnp.float32)
            cum = jax.lax.dot(ltl, cf, precision=jax.lax.Precision.HIGHEST)
            skey = jnp.where(hit, cum - cf, jnp.float32(-1.0))
            scr2[c, :] = cum[CHUNK - 1, :]

            colf = col_id.astype(jnp.float32)
            mx = jnp.max(cum[CHUNK - 1, :])
            for s in range(KSLOT):
                used = jnp.float32(s) < mx

                @pl.when(used)
                def _slot(c=c, s=s):
                    m = skey == float(s)
                    jval = jnp.sum(jnp.where(m, colf, 0.0), axis=0)
                    scr[c * KSLOT + s, :] = jval

                @pl.when(jnp.logical_not(used))
                def _slot0(c=c, s=s):
                    scr[c * KSLOT + s, :] = jnp.zeros((RB,), jnp.float32)

        @pl.when(jnp.logical_not(live))
        def _zero(c=c):
            scr[c * KSLOT:(c + 1) * KSLOT, :] = jnp.zeros(
                (KSLOT, RB), jnp.float32)
            scr2[c, :] = jnp.zeros((RB,), jnp.float32)

    jtab_ref[...] = scr[...].T.astype(jnp.int32)
    cnt_ref[...] = scr2[...].T.astype(jnp.int32)


def _passa(xmin, ymin, xmax, ymax):
    grid = (NP // RB,)
    return pl.pallas_call(
        _passa_body,
        grid=grid,
        in_specs=[
            pl.BlockSpec((1, RB), lambda i: (0, i)),
            pl.BlockSpec((1, RB), lambda i: (0, i)),
            pl.BlockSpec((1, RB), lambda i: (0, i)),
            pl.BlockSpec((1, RB), lambda i: (0, i)),
            pl.BlockSpec((NP, 1), lambda i: (0, 0)),
            pl.BlockSpec((NP, 1), lambda i: (0, 0)),
            pl.BlockSpec((NP, 1), lambda i: (0, 0)),
            pl.BlockSpec((NP, 1), lambda i: (0, 0)),
        ],
        out_specs=[pl.BlockSpec((RB, SLOTS), lambda i: (i, 0)),
                   pl.BlockSpec((RB, CPAD), lambda i: (i, 0))],
        out_shape=[jax.ShapeDtypeStruct((NP, SLOTS), jnp.int32),
                   jax.ShapeDtypeStruct((NP, CPAD), jnp.int32)],
        scratch_shapes=[pltpu.VMEM((SLOTS, RB), jnp.float32),
                        pltpu.VMEM((CPAD, RB), jnp.float32)],
    )(xmin.reshape(1, NP), ymin.reshape(1, NP),
      xmax.reshape(1, NP), ymax.reshape(1, NP),
      xmin.reshape(NP, 1), ymin.reshape(NP, 1),
      xmax.reshape(NP, 1), ymax.reshape(NP, 1))


def _iota16():
    return lax.iota(jnp.int32, 16)


def _splat(x):
    return jnp.full((16,), x, jnp.int32)


def _sc_body(jtab, cnt, xmin_h, ymin_h, xmax_h, ymax_h, scores_h, zeros_h,
             out_h,
             jbuf, cbuf, clist, xmin, ymin, xmax, ymax, scores,
             cand_i, cand_j, cand_d, delta, acc, stage, rdbk,
             sh_cnt, sh_delta):
    cid = lax.axis_index("c")
    tid = lax.axis_index("s")

    @pl.when(cid == 0)
    def _work():
        it16 = _iota16()
        rbase = tid * TPR

        pltpu.sync_copy(xmin_h, xmin)
        pltpu.sync_copy(ymin_h, ymin)
        pltpu.sync_copy(xmax_h, xmax)
        pltpu.sync_copy(ymax_h, ymax)
        pltpu.sync_copy(scores_h, scores)
        pltpu.sync_copy(zeros_h, delta)

        def _zb(k, _):
            z = jnp.zeros((16,), jnp.int32)
            cand_i[pl.ds(k * 16, 16)] = z
            cand_j[pl.ds(k * 16, 16)] = z
            return 0
        lax.fori_loop(0, NVREG, _zb, 0)

        def _group(g, off):
            pltpu.sync_copy(jtab.at[pl.ds((rbase + g * GROUP) * SLOTS,
                                          GROUP * SLOTS)], jbuf)
            pltpu.sync_copy(cnt.at[pl.ds((rbase + g * GROUP) * CPAD,
                                         GROUP * CPAD)], cbuf)

            def _row(rb, off):
                row_i = rbase + g * GROUP + rb
                cb = _splat(rb * CPAD)
                nc = jnp.int32(0)
                for q in range(CPAD // 16):
                    cq = plsc.load_gather(cbuf, [cb + _splat(q * 16) + it16])
                    mq = cq > 0
                    mqi = mq.astype(jnp.int32)
                    posq = _splat(nc) + plsc.cumsum(mqi) - mqi
                    plsc.store_scatter(clist, [posq], _splat(q * 16) + it16,
                                       mask=mq)
                    nc = nc + jnp.sum(mqi)

                def _chunk(ci, off):
                    c = jnp.max(plsc.load_gather(clist, [_splat(ci)]))
                    v = plsc.load_gather(
                        jbuf, [_splat(rb * SLOTS + c * 16) + it16])
                    m = v > 0
                    mi = m.astype(jnp.int32)
                    pos = _splat(off) + plsc.cumsum(mi) - mi
                    plsc.store_scatter(cand_j, [pos], v, mask=m)
                    plsc.store_scatter(cand_i, [pos], _splat(row_i), mask=m)
                    return jnp.minimum(off + jnp.sum(mi), CANDCAP - 16)
                return lax.fori_loop(0, nc, _chunk, off)
            return lax.fori_loop(0, GROUP, _row, off)

        t_cnt = lax.fori_loop(0, NGROUP, _group, jnp.int32(0))

        def _publish(val):
            stage[...] = _splat(val)
            pltpu.sync_copy(stage, sh_cnt.at[pl.ds(tid * 16, 16)])
            plsc.subcore_barrier()
            pltpu.sync_copy(sh_cnt, rdbk)

            def _acc(u, bt):
                base, tot = bt
                cu = jnp.max(plsc.load_gather(rdbk, [_splat(u) * _splat(16) + it16]))
                base = base + jnp.where(u < tid, cu, 0)
                return (base, tot + cu)
            base, tot = lax.fori_loop(0, NTILE, _acc,
                                      (jnp.int32(0), jnp.int32(0)))
            plsc.subcore_barrier()
            return base, tot

        base_t, _tot = _publish(t_cnt)
        m_t = jnp.clip(CAP - base_t, 0, t_cnt)
        nvd = (t_cnt + 15) // 16

        def _depth(k, _):
            lanes = _splat(k * 16) + it16
            valid = lanes < m_t
            ii_ = cand_i[pl.ds(k * 16, 16)]
            jj_ = cand_j[pl.ds(k * 16, 16)]
            axmin = plsc.load_gather(xmin, [ii_])
            axmax = plsc.load_gather(xmax, [ii_])
            bxmin = plsc.load_gather(xmin, [jj_])
            bxmax = plsc.load_gather(xmax, [jj_])
            aymin = plsc.load_gather(ymin, [ii_])
            aymax = plsc.load_gather(ymax, [ii_])
            bymin = plsc.load_gather(ymin, [jj_])
            bymax = plsc.load_gather(ymax, [jj_])
            ox = jnp.minimum(axmax, bxmax) - jnp.maximum(axmin, bxmin)
            oy = jnp.minimum(aymax, bymax) - jnp.maximum(aymin, bymin)
            d = jnp.minimum(ox, oy)
            d = jnp.where(valid & (d > 0), d, jnp.float32(-1.0))
            cand_d[pl.ds(k * 16, 16)] = d
            return 0
        lax.fori_loop(0, nvd, _depth, 0)

        def _count_ge(thr):
            def _cnt(k, c):
                d = cand_d[pl.ds(k * 16, 16)]
                di = plsc.bitcast(d, jnp.int32)
                return c + jnp.sum((di >= thr).astype(jnp.int32))
            local = lax.fori_loop(0, nvd, _cnt, jnp.int32(0))
            _, tot = _publish(local)
            return tot

        def _bs(_, lh):
            lo, hi = lh
            mid = (lo + hi) // 2
            c = _count_ge(mid)
            take = c >= KEEP
            return (jnp.where(take, mid, lo), jnp.where(take, hi, mid))
        lo, _hi = lax.fori_loop(0, 31, _bs,
                                (jnp.int32(1), jnp.int32(0x40C00002)))

        n_gt = _count_ge(lo + 1)
        extra = KEEP - n_gt

        def _ecnt(k, c):
            d = cand_d[pl.ds(k * 16, 16)]
            di = plsc.bitcast(d, jnp.int32)
            return c + jnp.sum((di == lo).astype(jnp.int32))
        e_t = lax.fori_loop(0, nvd, _ecnt, jnp.int32(0))
        base_e, _te = _publish(e_t)
        k_t = jnp.clip(extra - base_e, 0, e_t)

        def _sel(k, eqrun):
            d = cand_d[pl.ds(k * 16, 16)]
            di = plsc.bitcast(d, jnp.int32)
            sel_gt = di >= (lo + 1)
            meq = di == lo
            mi = meq.astype(jnp.int32)
            eqpos = _splat(eqrun) + plsc.cumsum(mi) - mi
            sel = sel_gt | (meq & (eqpos < k_t))
            ii_ = cand_i[pl.ds(k * 16, 16)]
            jj_ = cand_j[pl.ds(k * 16, 16)]
            axmin = plsc.load_gather(xmin, [ii_])
            axmax = plsc.load_gather(xmax, [ii_])
            bxmin = plsc.load_gather(xmin, [jj_])
            bxmax = plsc.load_gather(xmax, [jj_])
            aymin = plsc.load_gather(ymin, [ii_])
            aymax = plsc.load_gather(ymax, [ii_])
            bymin = plsc.load_gather(ymin, [jj_])
            bymax = plsc.load_gather(ymax, [jj_])
            ox = jnp.minimum(axmax, bxmax) - jnp.maximum(axmin, bxmin)
            oy = jnp.minimum(aymax, bymax) - jnp.maximum(aymin, bymin)
            cxa = (axmin + axmax) * 0.5
            cya = (aymin + aymax) * 0.5
            cxb = (bxmin + bxmax) * 0.5
            cyb = (bymin + bymax) * 0.5
            one = jnp.full((16,), 1.0, jnp.float32)
            sx = jnp.where(cxb >= cxa, one, -one)
            sy = jnp.where(cyb >= cya, one, -one)
            use_x = ox < oy
            zero = jnp.zeros((16,), jnp.float32)
            px = jnp.where(use_x, sx * ox, zero)
            py = jnp.where(use_x, zero, sy * oy)
            wi = plsc.load_gather(scores, [ii_])
            wj = plsc.load_gather(scores, [jj_])
            wsum = wi + wj
            mf = jnp.where(sel, one, zero)
            fi = wj / wsum * mf
            fj = wi / wsum * mf
            dix = -px * fi
            diy = -py * fi
            djx = px * fj
            djy = py * fj
            four = _splat(4)
            ibase = ii_ * four
            jbase = jj_ * four
            for col, val in ((0, dix), (1, diy), (2, dix), (3, diy)):
                plsc.addupdate_scatter(delta, [ibase + _splat(col)], val)
            for col, val in ((0, djx), (1, djy), (2, djx), (3, djy)):
                plsc.addupdate_scatter(delta, [jbase + _splat(col)], val)
            return eqrun + jnp.sum(mi)
        lax.fori_loop(0, nvd, _sel, jnp.int32(0))

        pltpu.sync_copy(delta, sh_delta.at[pl.ds(tid * NP * 4, NP * 4)])
        plsc.subcore_barrier()
        pltpu.sync_copy(zeros_h.at[pl.ds(0, TPR * 4)], acc)

        def _red(u, _):
            pltpu.sync_copy(sh_delta.at[pl.ds(u * NP * 4 + rbase * 4, TPR * 4)],
                            delta.at[pl.ds(0, TPR * 4)])

            def _addv(k, _):
                cur = acc[pl.ds(k * 16, 16)]
                add = delta[pl.ds(k * 16, 16)]
                acc[pl.ds(k * 16, 16)] = cur + add
                return 0
            lax.fori_loop(0, TPR * 4 // 16, _addv, 0)
            return 0
        lax.fori_loop(0, NTILE, _red, 0)

        pltpu.sync_copy(acc, out_h.at[pl.ds(rbase * 4, TPR * 4)])
        plsc.subcore_barrier()


def _passb(jtab, cnt, xmin, ymin, xmax, ymax, scores_p, zeros4):
    mesh = plsc.VectorSubcoreMesh(core_axis_name="c", subcore_axis_name="s")
    f32 = jnp.float32
    kern = functools.partial(
        pl.kernel,
        mesh=mesh,
        compiler_params=pltpu.CompilerParams(needs_layout_passes=False),
        out_type=jax.ShapeDtypeStruct((NP * 4,), f32),
        scratch_types=[
            pltpu.VMEM((GROUP * SLOTS,), jnp.int32),
            pltpu.VMEM((GROUP * CPAD,), jnp.int32),
            pltpu.VMEM((64,), jnp.int32),
            pltpu.VMEM((NP,), f32),
            pltpu.VMEM((NP,), f32),
            pltpu.VMEM((NP,), f32),
            pltpu.VMEM((NP,), f32),
            pltpu.VMEM((NP,), f32),
            pltpu.VMEM((CANDCAP,), jnp.int32),
            pltpu.VMEM((CANDCAP,), jnp.int32),
            pltpu.VMEM((CANDCAP,), f32),
            pltpu.VMEM((NP * 4,), f32),
            pltpu.VMEM((TPR * 4,), f32),
            pltpu.VMEM((16,), jnp.int32),
            pltpu.VMEM((NTILE * 16,), jnp.int32),
            pltpu.VMEM_SHARED((NTILE * 16,), jnp.int32),
            pltpu.VMEM_SHARED((NTILE * NP * 4,), f32),
        ],
    )
    return kern(_sc_body)(jtab, cnt, xmin, ymin, xmax, ymax, scores_p, zeros4)


def _passc_body(b_ref, d_ref, o_ref):
    o_ref[...] = b_ref[...] + d_ref[...]


def _passc(boxes_p, delta):
    return pl.pallas_call(
        _passc_body,
        out_shape=jax.ShapeDtypeStruct((NP, 4), jnp.float32),
    )(boxes_p, delta)


def _pad_cols(boxes):
    pad = NP - N
    xmin = jnp.pad(boxes[:, 0], (0, pad), constant_values=3.0e30)
    ymin = jnp.pad(boxes[:, 1], (0, pad), constant_values=3.0e30)
    xmax = jnp.pad(boxes[:, 2], (0, pad), constant_values=-3.0e30)
    ymax = jnp.pad(boxes[:, 3], (0, pad), constant_values=-3.0e30)
    return xmin, ymin, xmax, ymax


def kernel(boxes, scores):
    xmin, ymin, xmax, ymax = _pad_cols(boxes)
    jtab, cnt = _passa(xmin, ymin, xmax, ymax)
    scores_p = jnp.pad(scores, (0, NP - N))
    zeros4 = jnp.zeros((NP * 4,), jnp.float32)
    delta = _passb(jtab.reshape(NP * SLOTS), cnt.reshape(NP * CPAD),
                   xmin, ymin, xmax, ymax, scores_p, zeros4)
    boxes_p = jnp.pad(boxes, ((0, NP - N), (0, 0)))
    out = _passc(boxes_p, delta.reshape(NP, 4))
    return out[:N]


# speedup vs baseline: 2.3336x; 1.1285x over previous
import functools

import jax
import jax.numpy as jnp
from jax import lax
from jax.experimental import pallas as pl
from jax.experimental.pallas import tpu as pltpu
from jax.experimental.pallas import tpu_sc as plsc

N = 5000
NP = 5120
CHUNK = 128
KSLOT = 16
C = NP // CHUNK
RB = 512
CAP = 4 * N
KEEP = N

NTILE = 16
TPR = NP // NTILE
GROUP = 32
NGROUP = TPR // GROUP
CANDCAP = 4096
NVREG = CANDCAP // 16
SLOTS = C * KSLOT
SVREG = SLOTS // 16
CPAD = 48


def _passa_body(xmin_r, ymin_r, xmax_r, ymax_r,
                xmin_c, ymin_c, xmax_c, ymax_c, jtab_ref, cnt_ref, scr, scr2):
    ri = pl.program_id(0)
    a_xmin = xmin_r[...]
    a_ymin = ymin_r[...]
    a_xmax = xmax_r[...]
    a_ymax = ymax_r[...]

    row_id = ri * RB + lax.broadcasted_iota(jnp.int32, (CHUNK, RB), 1)

    ii = lax.broadcasted_iota(jnp.int32, (CHUNK, CHUNK), 0)
    jj = lax.broadcasted_iota(jnp.int32, (CHUNK, CHUNK), 1)
    ltl = (jj <= ii).astype(jnp.bfloat16)
    scr2[C:CPAD, :] = jnp.zeros((CPAD - C, RB), jnp.float32)

    for c in range(C):
        live = ri * (RB // CHUNK) <= c

        @pl.when(live)
        def _compute(c=c):
            sl = slice(c * CHUNK, (c + 1) * CHUNK)
            b_xmin = xmin_c[sl, :]
            b_ymin = ymin_c[sl, :]
            b_xmax = xmax_c[sl, :]
            b_ymax = ymax_c[sl, :]
            col_id = c * CHUNK + lax.broadcasted_iota(jnp.int32, (CHUNK, RB), 0)

            hit = ((a_xmin <= b_xmax) & (b_xmin <= a_xmax)
                   & (a_ymin <= b_ymax) & (b_ymin <= a_ymax)
                   & (col_id > row_id))
            cf = hit.astype(jnp.bfloat16)
            cum = jax.lax.dot(ltl, cf,
                              preferred_element_type=jnp.float32)
            cf = cf.astype(jnp.float32)
            skey = jnp.where(hit, cum - cf, jnp.float32(-1.0))
            scr2[c, :] = cum[CHUNK - 1, :]

            colf = col_id.astype(jnp.float32)
            mx = jnp.max(cum[CHUNK - 1, :])
            for s in range(KSLOT):
                used = jnp.float32(s) < mx

                @pl.when(used)
                def _slot(c=c, s=s):
                    m = skey == float(s)
                    jval = jnp.sum(jnp.where(m, colf, 0.0), axis=0)
                    scr[c * KSLOT + s, :] = jval

                @pl.when(jnp.logical_not(used))
                def _slot0(c=c, s=s):
                    scr[c * KSLOT + s, :] = jnp.zeros((RB,), jnp.float32)

        @pl.when(jnp.logical_not(live))
        def _zero(c=c):
            scr[c * KSLOT:(c + 1) * KSLOT, :] = jnp.zeros(
                (KSLOT, RB), jnp.float32)
            scr2[c, :] = jnp.zeros((RB,), jnp.float32)

    jtab_ref[...] = scr[...].T.astype(jnp.int32)
    cnt_ref[...] = scr2[...].T.astype(jnp.int32)


def _passa(xmin, ymin, xmax, ymax):
    grid = (NP // RB,)
    return pl.pallas_call(
        _passa_body,
        grid=grid,
        in_specs=[
            pl.BlockSpec((1, RB), lambda i: (0, i)),
            pl.BlockSpec((1, RB), lambda i: (0, i)),
            pl.BlockSpec((1, RB), lambda i: (0, i)),
            pl.BlockSpec((1, RB), lambda i: (0, i)),
            pl.BlockSpec((NP, 1), lambda i: (0, 0)),
            pl.BlockSpec((NP, 1), lambda i: (0, 0)),
            pl.BlockSpec((NP, 1), lambda i: (0, 0)),
            pl.BlockSpec((NP, 1), lambda i: (0, 0)),
        ],
        out_specs=[pl.BlockSpec((RB, SLOTS), lambda i: (i, 0)),
                   pl.BlockSpec((RB, CPAD), lambda i: (i, 0))],
        out_shape=[jax.ShapeDtypeStruct((NP, SLOTS), jnp.int32),
                   jax.ShapeDtypeStruct((NP, CPAD), jnp.int32)],
        scratch_shapes=[pltpu.VMEM((SLOTS, RB), jnp.float32),
                        pltpu.VMEM((CPAD, RB), jnp.float32)],
    )(xmin.reshape(1, NP), ymin.reshape(1, NP),
      xmax.reshape(1, NP), ymax.reshape(1, NP),
      xmin.reshape(NP, 1), ymin.reshape(NP, 1),
      xmax.reshape(NP, 1), ymax.reshape(NP, 1))


def _iota16():
    return lax.iota(jnp.int32, 16)


def _splat(x):
    return jnp.full((16,), x, jnp.int32)


def _sc_body(jtab, cnt, xmin_h, ymin_h, xmax_h, ymax_h, scores_h, zeros_h,
             out_h,
             jbuf, cbuf, clist, xmin, ymin, xmax, ymax, scores,
             cand_i, cand_j, cand_d, delta, acc, stage, rdbk,
             sh_cnt, sh_delta):
    cid = lax.axis_index("c")
    tid = lax.axis_index("s")

    @pl.when(cid == 0)
    def _work():
        it16 = _iota16()
        rbase = tid * TPR

        pltpu.sync_copy(xmin_h, xmin)
        pltpu.sync_copy(ymin_h, ymin)
        pltpu.sync_copy(xmax_h, xmax)
        pltpu.sync_copy(ymax_h, ymax)
        pltpu.sync_copy(scores_h, scores)
        pltpu.sync_copy(zeros_h, delta)

        def _zb(k, _):
            z = jnp.zeros((16,), jnp.int32)
            cand_i[pl.ds(k * 16, 16)] = z
            cand_j[pl.ds(k * 16, 16)] = z
            return 0
        lax.fori_loop(0, NVREG, _zb, 0)

        def _group(g, off):
            pltpu.sync_copy(jtab.at[pl.ds((rbase + g * GROUP) * SLOTS,
                                          GROUP * SLOTS)], jbuf)
            pltpu.sync_copy(cnt.at[pl.ds((rbase + g * GROUP) * CPAD,
                                         GROUP * CPAD)], cbuf)

            def _row(rb, off):
                row_i = rbase + g * GROUP + rb
                cb = _splat(rb * CPAD)
                nc = jnp.int32(0)
                for q in range(CPAD // 16):
                    cq = plsc.load_gather(cbuf, [cb + _splat(q * 16) + it16])
                    mq = cq > 0
                    mqi = mq.astype(jnp.int32)
                    posq = _splat(nc) + plsc.cumsum(mqi) - mqi
                    plsc.store_scatter(clist, [posq], _splat(q * 16) + it16,
                                       mask=mq)
                    nc = nc + jnp.sum(mqi)

                def _chunk(ci, off):
                    c = jnp.max(plsc.load_gather(clist, [_splat(ci)]))
                    v = plsc.load_gather(
                        jbuf, [_splat(rb * SLOTS + c * 16) + it16])
                    m = v > 0
                    mi = m.astype(jnp.int32)
                    pos = _splat(off) + plsc.cumsum(mi) - mi
                    plsc.store_scatter(cand_j, [pos], v, mask=m)
                    plsc.store_scatter(cand_i, [pos], _splat(row_i), mask=m)
                    return jnp.minimum(off + jnp.sum(mi), CANDCAP - 16)
                return lax.fori_loop(0, nc, _chunk, off)
            return lax.fori_loop(0, GROUP, _row, off)

        t_cnt = lax.fori_loop(0, NGROUP, _group, jnp.int32(0))

        def _publish(val):
            stage[...] = _splat(val)
            pltpu.sync_copy(stage, sh_cnt.at[pl.ds(tid * 16, 16)])
            plsc.subcore_barrier()
            pltpu.sync_copy(sh_cnt, rdbk)

            def _acc(u, bt):
                base, tot = bt
                cu = jnp.max(plsc.load_gather(rdbk, [_splat(u) * _splat(16) + it16]))
                base = base + jnp.where(u < tid, cu, 0)
                return (base, tot + cu)
            base, tot = lax.fori_loop(0, NTILE, _acc,
                                      (jnp.int32(0), jnp.int32(0)))
            plsc.subcore_barrier()
            return base, tot

        base_t, _tot = _publish(t_cnt)
        m_t = jnp.clip(CAP - base_t, 0, t_cnt)
        nvd = (t_cnt + 15) // 16

        def _depth(k, _):
            lanes = _splat(k * 16) + it16
            valid = lanes < m_t
            ii_ = cand_i[pl.ds(k * 16, 16)]
            jj_ = cand_j[pl.ds(k * 16, 16)]
            axmin = plsc.load_gather(xmin, [ii_])
            axmax = plsc.load_gather(xmax, [ii_])
            bxmin = plsc.load_gather(xmin, [jj_])
            bxmax = plsc.load_gather(xmax, [jj_])
            aymin = plsc.load_gather(ymin, [ii_])
            aymax = plsc.load_gather(ymax, [ii_])
            bymin = plsc.load_gather(ymin, [jj_])
            bymax = plsc.load_gather(ymax, [jj_])
            ox = jnp.minimum(axmax, bxmax) - jnp.maximum(axmin, bxmin)
            oy = jnp.minimum(aymax, bymax) - jnp.maximum(aymin, bymin)
            d = jnp.minimum(ox, oy)
            d = jnp.where(valid & (d > 0), d, jnp.float32(-1.0))
            cand_d[pl.ds(k * 16, 16)] = d
            return 0
        lax.fori_loop(0, nvd, _depth, 0)

        def _count_ge(thr):
            def _cnt(k, c):
                d = cand_d[pl.ds(k * 16, 16)]
                di = plsc.bitcast(d, jnp.int32)
                return c + jnp.sum((di >= thr).astype(jnp.int32))
            local = lax.fori_loop(0, nvd, _cnt, jnp.int32(0))
            _, tot = _publish(local)
            return tot

        def _bs(_, lh):
            lo, hi = lh
            mid = (lo + hi) // 2
            c = _count_ge(mid)
            take = c >= KEEP
            return (jnp.where(take, mid, lo), jnp.where(take, hi, mid))
        lo, _hi = lax.fori_loop(0, 31, _bs,
                                (jnp.int32(1), jnp.int32(0x40C00002)))

        n_gt = _count_ge(lo + 1)
        extra = KEEP - n_gt

        def _ecnt(k, c):
            d = cand_d[pl.ds(k * 16, 16)]
            di = plsc.bitcast(d, jnp.int32)
            return c + jnp.sum((di == lo).astype(jnp.int32))
        e_t = lax.fori_loop(0, nvd, _ecnt, jnp.int32(0))
        base_e, _te = _publish(e_t)
        k_t = jnp.clip(extra - base_e, 0, e_t)

        def _sel(k, eqrun):
            d = cand_d[pl.ds(k * 16, 16)]
            di = plsc.bitcast(d, jnp.int32)
            sel_gt = di >= (lo + 1)
            meq = di == lo
            mi = meq.astype(jnp.int32)
            eqpos = _splat(eqrun) + plsc.cumsum(mi) - mi
            sel = sel_gt | (meq & (eqpos < k_t))
            ii_ = cand_i[pl.ds(k * 16, 16)]
            jj_ = cand_j[pl.ds(k * 16, 16)]
            axmin = plsc.load_gather(xmin, [ii_])
            axmax = plsc.load_gather(xmax, [ii_])
            bxmin = plsc.load_gather(xmin, [jj_])
            bxmax = plsc.load_gather(xmax, [jj_])
            aymin = plsc.load_gather(ymin, [ii_])
            aymax = plsc.load_gather(ymax, [ii_])
            bymin = plsc.load_gather(ymin, [jj_])
            bymax = plsc.load_gather(ymax, [jj_])
            ox = jnp.minimum(axmax, bxmax) - jnp.maximum(axmin, bxmin)
            oy = jnp.minimum(aymax, bymax) - jnp.maximum(aymin, bymin)
            cxa = (axmin + axmax) * 0.5
            cya = (aymin + aymax) * 0.5
            cxb = (bxmin + bxmax) * 0.5
            cyb = (bymin + bymax) * 0.5
            one = jnp.full((16,), 1.0, jnp.float32)
            sx = jnp.where(cxb >= cxa, one, -one)
            sy = jnp.where(cyb >= cya, one, -one)
            use_x = ox < oy
            zero = jnp.zeros((16,), jnp.float32)
            px = jnp.where(use_x, sx * ox, zero)
            py = jnp.where(use_x, zero, sy * oy)
            wi = plsc.load_gather(scores, [ii_])
            wj = plsc.load_gather(scores, [jj_])
            wsum = wi + wj
            mf = jnp.where(sel, one, zero)
            fi = wj / wsum * mf
            fj = wi / wsum * mf
            dix = -px * fi
            diy = -py * fi
            djx = px * fj
            djy = py * fj
            four = _splat(4)
            ibase = ii_ * four
            jbase = jj_ * four
            for col, val in ((0, dix), (1, diy), (2, dix), (3, diy)):
                plsc.addupdate_scatter(delta, [ibase + _splat(col)], val)
            for col, val in ((0, djx), (1, djy), (2, djx), (3, djy)):
                plsc.addupdate_scatter(delta, [jbase + _splat(col)], val)
            return eqrun + jnp.sum(mi)
        lax.fori_loop(0, nvd, _sel, jnp.int32(0))

        pltpu.sync_copy(delta, sh_delta.at[pl.ds(tid * NP * 4, NP * 4)])
        plsc.subcore_barrier()
        pltpu.sync_copy(zeros_h.at[pl.ds(0, TPR * 4)], acc)

        def _red(u, _):
            pltpu.sync_copy(sh_delta.at[pl.ds(u * NP * 4 + rbase * 4, TPR * 4)],
                            delta.at[pl.ds(0, TPR * 4)])

            def _addv(k, _):
                cur = acc[pl.ds(k * 16, 16)]
                add = delta[pl.ds(k * 16, 16)]
                acc[pl.ds(k * 16, 16)] = cur + add
                return 0
            lax.fori_loop(0, TPR * 4 // 16, _addv, 0)
            return 0
        lax.fori_loop(0, NTILE, _red, 0)

        pltpu.sync_copy(acc, out_h.at[pl.ds(rbase * 4, TPR * 4)])
        plsc.subcore_barrier()


def _passb(jtab, cnt, xmin, ymin, xmax, ymax, scores_p, zeros4):
    mesh = plsc.VectorSubcoreMesh(core_axis_name="c", subcore_axis_name="s")
    f32 = jnp.float32
    kern = functools.partial(
        pl.kernel,
        mesh=mesh,
        compiler_params=pltpu.CompilerParams(needs_layout_passes=False),
        out_type=jax.ShapeDtypeStruct((NP * 4,), f32),
        scratch_types=[
            pltpu.VMEM((GROUP * SLOTS,), jnp.int32),
            pltpu.VMEM((GROUP * CPAD,), jnp.int32),
            pltpu.VMEM((64,), jnp.int32),
            pltpu.VMEM((NP,), f32),
            pltpu.VMEM((NP,), f32),
            pltpu.VMEM((NP,), f32),
            pltpu.VMEM((NP,), f32),
            pltpu.VMEM((NP,), f32),
            pltpu.VMEM((CANDCAP,), jnp.int32),
            pltpu.VMEM((CANDCAP,), jnp.int32),
            pltpu.VMEM((CANDCAP,), f32),
            pltpu.VMEM((NP * 4,), f32),
            pltpu.VMEM((TPR * 4,), f32),
            pltpu.VMEM((16,), jnp.int32),
            pltpu.VMEM((NTILE * 16,), jnp.int32),
            pltpu.VMEM_SHARED((NTILE * 16,), jnp.int32),
            pltpu.VMEM_SHARED((NTILE * NP * 4,), f32),
        ],
    )
    return kern(_sc_body)(jtab, cnt, xmin, ymin, xmax, ymax, scores_p, zeros4)


def _passc_body(b_ref, d_ref, o_ref):
    o_ref[...] = b_ref[...] + d_ref[...]


def _passc(boxes_p, delta):
    return pl.pallas_call(
        _passc_body,
        out_shape=jax.ShapeDtypeStruct((NP, 4), jnp.float32),
    )(boxes_p, delta)


def _pad_cols(boxes):
    pad = NP - N
    xmin = jnp.pad(boxes[:, 0], (0, pad), constant_values=3.0e30)
    ymin = jnp.pad(boxes[:, 1], (0, pad), constant_values=3.0e30)
    xmax = jnp.pad(boxes[:, 2], (0, pad), constant_values=-3.0e30)
    ymax = jnp.pad(boxes[:, 3], (0, pad), constant_values=-3.0e30)
    return xmin, ymin, xmax, ymax


def kernel(boxes, scores):
    xmin, ymin, xmax, ymax = _pad_cols(boxes)
    jtab, cnt = _passa(xmin, ymin, xmax, ymax)
    scores_p = jnp.pad(scores, (0, NP - N))
    zeros4 = jnp.zeros((NP * 4,), jnp.float32)
    delta = _passb(jtab.reshape(NP * SLOTS), cnt.reshape(NP * CPAD),
                   xmin, ymin, xmax, ymax, scores_p, zeros4)
    boxes_p = jnp.pad(boxes, ((0, NP - N), (0, 0)))
    out = _passc(boxes_p, delta.reshape(NP, 4))
    return out[:N]


# speedup vs baseline: 3.3941x; 1.4545x over previous
import functools

import jax
import jax.numpy as jnp
from jax import lax
from jax.experimental import pallas as pl
from jax.experimental.pallas import tpu as pltpu
from jax.experimental.pallas import tpu_sc as plsc

N = 5000
NP = 5120
CHUNK = 128
KSLOT = 16
C = NP // CHUNK
RB = 512
CAP = 4 * N
KEEP = N

NTILE = 16
TPR = NP // NTILE
GROUP = 32
NGROUP = TPR // GROUP
CANDCAP = 4096
NVREG = CANDCAP // 16
SLOTS = C * KSLOT
SVREG = SLOTS // 16
CPAD = 48


def _passa_body(xmin_r, ymin_r, xmax_r, ymax_r,
                xmin_c, ymin_c, xmax_c, ymax_c, jtab_ref, cnt_ref, scr, scr2):
    ri = pl.program_id(0)
    a_xmin = xmin_r[...]
    a_ymin = ymin_r[...]
    a_xmax = xmax_r[...]
    a_ymax = ymax_r[...]

    row_id = ri * RB + lax.broadcasted_iota(jnp.int32, (CHUNK, RB), 1)

    ii = lax.broadcasted_iota(jnp.int32, (CHUNK, CHUNK), 0)
    jj = lax.broadcasted_iota(jnp.int32, (CHUNK, CHUNK), 1)
    ltl = (jj <= ii).astype(jnp.bfloat16)
    scr2[C:CPAD, :] = jnp.zeros((CPAD - C, RB), jnp.float32)

    for c in range(C):
        live = ri * (RB // CHUNK) <= c

        @pl.when(live)
        def _compute(c=c):
            sl = slice(c * CHUNK, (c + 1) * CHUNK)
            b_xmin = xmin_c[sl, :]
            b_ymin = ymin_c[sl, :]
            b_xmax = xmax_c[sl, :]
            b_ymax = ymax_c[sl, :]
            col_id = c * CHUNK + lax.broadcasted_iota(jnp.int32, (CHUNK, RB), 0)

            hit = ((a_xmin <= b_xmax) & (b_xmin <= a_xmax)
                   & (a_ymin <= b_ymax) & (b_ymin <= a_ymax)
                   & (col_id > row_id))
            cf = hit.astype(jnp.bfloat16)
            cum = jax.lax.dot(ltl, cf,
                              preferred_element_type=jnp.float32)
            cf = cf.astype(jnp.float32)
            skey = jnp.where(hit, cum - cf, jnp.float32(-1.0))
            scr2[c, :] = cum[CHUNK - 1, :]

            colf = (lax.broadcasted_iota(jnp.int32, (CHUNK, RB), 0)
                    + 1).astype(jnp.float32)
            mx = jnp.max(cum[CHUNK - 1, :])
            for s in range(KSLOT):
                used = jnp.float32(s) < mx

                @pl.when(used)
                def _slot(c=c, s=s):
                    m = skey == float(s)
                    jval = jnp.sum(jnp.where(m, colf, 0.0), axis=0)
                    scr[c * KSLOT + s, :] = jval

                @pl.when(jnp.logical_not(used))
                def _slot0(c=c, s=s):
                    scr[c * KSLOT + s, :] = jnp.zeros((RB,), jnp.float32)

        @pl.when(jnp.logical_not(live))
        def _zero(c=c):
            scr[c * KSLOT:(c + 1) * KSLOT, :] = jnp.zeros(
                (KSLOT, RB), jnp.float32)
            scr2[c, :] = jnp.zeros((RB,), jnp.float32)

    s1 = lax.broadcasted_iota(jnp.int32, (SLOTS, SLOTS), 0)
    s2 = lax.broadcasted_iota(jnp.int32, (SLOTS, SLOTS), 1)
    eye_s = (s1 == s2).astype(jnp.bfloat16)
    t1 = lax.broadcasted_iota(jnp.int32, (CPAD, CPAD), 0)
    t2 = lax.broadcasted_iota(jnp.int32, (CPAD, CPAD), 1)
    eye_c = (t1 == t2).astype(jnp.bfloat16)
    jt = lax.dot_general(scr[...].astype(jnp.bfloat16), eye_s,
                         (((0,), (0,)), ((), ())),
                         preferred_element_type=jnp.float32)
    ct = lax.dot_general(scr2[...].astype(jnp.bfloat16), eye_c,
                         (((0,), (0,)), ((), ())),
                         preferred_element_type=jnp.float32)
    jtab_ref[...] = jt.astype(jnp.int32)
    cnt_ref[...] = ct.astype(jnp.int32)


def _passa(xmin, ymin, xmax, ymax):
    grid = (NP // RB,)
    return pl.pallas_call(
        _passa_body,
        grid=grid,
        in_specs=[
            pl.BlockSpec((1, RB), lambda i: (0, i)),
            pl.BlockSpec((1, RB), lambda i: (0, i)),
            pl.BlockSpec((1, RB), lambda i: (0, i)),
            pl.BlockSpec((1, RB), lambda i: (0, i)),
            pl.BlockSpec((NP, 1), lambda i: (0, 0)),
            pl.BlockSpec((NP, 1), lambda i: (0, 0)),
            pl.BlockSpec((NP, 1), lambda i: (0, 0)),
            pl.BlockSpec((NP, 1), lambda i: (0, 0)),
        ],
        out_specs=[pl.BlockSpec((RB, SLOTS), lambda i: (i, 0)),
                   pl.BlockSpec((RB, CPAD), lambda i: (i, 0))],
        out_shape=[jax.ShapeDtypeStruct((NP, SLOTS), jnp.int32),
                   jax.ShapeDtypeStruct((NP, CPAD), jnp.int32)],
        scratch_shapes=[pltpu.VMEM((SLOTS, RB), jnp.float32),
                        pltpu.VMEM((CPAD, RB), jnp.float32)],
    )(xmin.reshape(1, NP), ymin.reshape(1, NP),
      xmax.reshape(1, NP), ymax.reshape(1, NP),
      xmin.reshape(NP, 1), ymin.reshape(NP, 1),
      xmax.reshape(NP, 1), ymax.reshape(NP, 1))


def _iota16():
    return lax.iota(jnp.int32, 16)


def _splat(x):
    return jnp.full((16,), x, jnp.int32)


def _sc_body(jtab, cnt, xmin_h, ymin_h, xmax_h, ymax_h, scores_h, zeros_h,
             out_h,
             jbuf, cbuf, clist, xmin, ymin, xmax, ymax, scores,
             cand_i, cand_j, cand_d, delta, acc, stage, rdbk,
             sh_cnt, sh_delta):
    cid = lax.axis_index("c")
    tid = lax.axis_index("s")

    @pl.when(cid == 0)
    def _work():
        it16 = _iota16()
        rbase = tid * TPR

        pltpu.sync_copy(xmin_h, xmin)
        pltpu.sync_copy(ymin_h, ymin)
        pltpu.sync_copy(xmax_h, xmax)
        pltpu.sync_copy(ymax_h, ymax)
        pltpu.sync_copy(scores_h, scores)
        pltpu.sync_copy(zeros_h, delta)

        def _zb(k, _):
            z = jnp.zeros((16,), jnp.int32)
            cand_i[pl.ds(k * 16, 16)] = z
            cand_j[pl.ds(k * 16, 16)] = z
            return 0
        lax.fori_loop(0, NVREG, _zb, 0)

        def _group(g, off):
            pltpu.sync_copy(jtab.at[pl.ds((rbase + g * GROUP) * SLOTS,
                                          GROUP * SLOTS)], jbuf)
            pltpu.sync_copy(cnt.at[pl.ds((rbase + g * GROUP) * CPAD,
                                         GROUP * CPAD)], cbuf)

            def _row(rb, off):
                row_i = rbase + g * GROUP + rb
                cb = _splat(rb * CPAD)
                nc = jnp.int32(0)
                for q in range(CPAD // 16):
                    cq = plsc.load_gather(cbuf, [cb + _splat(q * 16) + it16])
                    mq = cq > 0
                    mqi = mq.astype(jnp.int32)
                    posq = _splat(nc) + plsc.cumsum(mqi) - mqi
                    plsc.store_scatter(clist, [posq], _splat(q * 16) + it16,
                                       mask=mq)
                    nc = nc + jnp.sum(mqi)

                def _chunk(ci, off):
                    c = jnp.max(plsc.load_gather(clist, [_splat(ci)]))
                    v = plsc.load_gather(
                        jbuf, [_splat(rb * SLOTS + c * 16) + it16])
                    m = v > 0
                    mi = m.astype(jnp.int32)
                    pos = _splat(off) + plsc.cumsum(mi) - mi
                    vj = v + _splat(c * CHUNK - 1)
                    plsc.store_scatter(cand_j, [pos], vj, mask=m)
                    plsc.store_scatter(cand_i, [pos], _splat(row_i), mask=m)
                    return jnp.minimum(off + jnp.sum(mi), CANDCAP - 16)
                return lax.fori_loop(0, nc, _chunk, off)
            return lax.fori_loop(0, GROUP, _row, off)

        t_cnt = lax.fori_loop(0, NGROUP, _group, jnp.int32(0))

        def _publish(val):
            stage[...] = _splat(val)
            pltpu.sync_copy(stage, sh_cnt.at[pl.ds(tid * 16, 16)])
            plsc.subcore_barrier()
            pltpu.sync_copy(sh_cnt, rdbk)

            def _acc(u, bt):
                base, tot = bt
                cu = jnp.max(plsc.load_gather(rdbk, [_splat(u) * _splat(16) + it16]))
                base = base + jnp.where(u < tid, cu, 0)
                return (base, tot + cu)
            base, tot = lax.fori_loop(0, NTILE, _acc,
                                      (jnp.int32(0), jnp.int32(0)))
            plsc.subcore_barrier()
            return base, tot

        base_t, _tot = _publish(t_cnt)
        m_t = jnp.clip(CAP - base_t, 0, t_cnt)
        nvd = (t_cnt + 15) // 16

        def _depth(k, _):
            lanes = _splat(k * 16) + it16
            valid = lanes < m_t
            ii_ = cand_i[pl.ds(k * 16, 16)]
            jj_ = cand_j[pl.ds(k * 16, 16)]
            axmin = plsc.load_gather(xmin, [ii_])
            axmax = plsc.load_gather(xmax, [ii_])
            bxmin = plsc.load_gather(xmin, [jj_])
            bxmax = plsc.load_gather(xmax, [jj_])
            aymin = plsc.load_gather(ymin, [ii_])
            aymax = plsc.load_gather(ymax, [ii_])
            bymin = plsc.load_gather(ymin, [jj_])
            bymax = plsc.load_gather(ymax, [jj_])
            ox = jnp.minimum(axmax, bxmax) - jnp.maximum(axmin, bxmin)
            oy = jnp.minimum(aymax, bymax) - jnp.maximum(aymin, bymin)
            d = jnp.minimum(ox, oy)
            d = jnp.where(valid & (d > 0), d, jnp.float32(-1.0))
            cand_d[pl.ds(k * 16, 16)] = d
            return 0
        lax.fori_loop(0, nvd, _depth, 0)

        def _count_ge(thr):
            def _cnt(k, c):
                d = cand_d[pl.ds(k * 16, 16)]
                di = plsc.bitcast(d, jnp.int32)
                return c + jnp.sum((di >= thr).astype(jnp.int32))
            local = lax.fori_loop(0, nvd, _cnt, jnp.int32(0))
            _, tot = _publish(local)
            return tot

        def _bs(_, lh):
            lo, hi = lh
            mid = (lo + hi) // 2
            c = _count_ge(mid)
            take = c >= KEEP
            return (jnp.where(take, mid, lo), jnp.where(take, hi, mid))
        lo, _hi = lax.fori_loop(0, 31, _bs,
                                (jnp.int32(1), jnp.int32(0x40C00002)))

        n_gt = _count_ge(lo + 1)
        extra = KEEP - n_gt

        def _ecnt(k, c):
            d = cand_d[pl.ds(k * 16, 16)]
            di = plsc.bitcast(d, jnp.int32)
            return c + jnp.sum((di == lo).astype(jnp.int32))
        e_t = lax.fori_loop(0, nvd, _ecnt, jnp.int32(0))
        base_e, _te = _publish(e_t)
        k_t = jnp.clip(extra - base_e, 0, e_t)

        def _sel(k, eqrun):
            d = cand_d[pl.ds(k * 16, 16)]
            di = plsc.bitcast(d, jnp.int32)
            sel_gt = di >= (lo + 1)
            meq = di == lo
            mi = meq.astype(jnp.int32)
            eqpos = _splat(eqrun) + plsc.cumsum(mi) - mi
            sel = sel_gt | (meq & (eqpos < k_t))
            ii_ = cand_i[pl.ds(k * 16, 16)]
            jj_ = cand_j[pl.ds(k * 16, 16)]
            axmin = plsc.load_gather(xmin, [ii_])
            axmax = plsc.load_gather(xmax, [ii_])
            bxmin = plsc.load_gather(xmin, [jj_])
            bxmax = plsc.load_gather(xmax, [jj_])
            aymin = plsc.load_gather(ymin, [ii_])
            aymax = plsc.load_gather(ymax, [ii_])
            bymin = plsc.load_gather(ymin, [jj_])
            bymax = plsc.load_gather(ymax, [jj_])
            ox = jnp.minimum(axmax, bxmax) - jnp.maximum(axmin, bxmin)
            oy = jnp.minimum(aymax, bymax) - jnp.maximum(aymin, bymin)
            cxa = (axmin + axmax) * 0.5
            cya = (aymin + aymax) * 0.5
            cxb = (bxmin + bxmax) * 0.5
            cyb = (bymin + bymax) * 0.5
            one = jnp.full((16,), 1.0, jnp.float32)
            sx = jnp.where(cxb >= cxa, one, -one)
            sy = jnp.where(cyb >= cya, one, -one)
            use_x = ox < oy
            zero = jnp.zeros((16,), jnp.float32)
            px = jnp.where(use_x, sx * ox, zero)
            py = jnp.where(use_x, zero, sy * oy)
            wi = plsc.load_gather(scores, [ii_])
            wj = plsc.load_gather(scores, [jj_])
            wsum = wi + wj
            mf = jnp.where(sel, one, zero)
            fi = wj / wsum * mf
            fj = wi / wsum * mf
            dix = -px * fi
            diy = -py * fi
            djx = px * fj
            djy = py * fj
            four = _splat(4)
            ibase = ii_ * four
            jbase = jj_ * four
            for col, val in ((0, dix), (1, diy), (2, dix), (3, diy)):
                plsc.addupdate_scatter(delta, [ibase + _splat(col)], val)
            for col, val in ((0, djx), (1, djy), (2, djx), (3, djy)):
                plsc.addupdate_scatter(delta, [jbase + _splat(col)], val)
            return eqrun + jnp.sum(mi)
        lax.fori_loop(0, nvd, _sel, jnp.int32(0))

        pltpu.sync_copy(delta, sh_delta.at[pl.ds(tid * NP * 4, NP * 4)])
        plsc.subcore_barrier()
        pltpu.sync_copy(zeros_h.at[pl.ds(0, TPR * 4)], acc)

        def _red(u, _):
            pltpu.sync_copy(sh_delta.at[pl.ds(u * NP * 4 + rbase * 4, TPR * 4)],
                            delta.at[pl.ds(0, TPR * 4)])

            def _addv(k, _):
                cur = acc[pl.ds(k * 16, 16)]
                add = delta[pl.ds(k * 16, 16)]
                acc[pl.ds(k * 16, 16)] = cur + add
                return 0
            lax.fori_loop(0, TPR * 4 // 16, _addv, 0)
            return 0
        lax.fori_loop(0, NTILE, _red, 0)

        pltpu.sync_copy(acc, out_h.at[pl.ds(rbase * 4, TPR * 4)])
        plsc.subcore_barrier()


def _passb(jtab, cnt, xmin, ymin, xmax, ymax, scores_p, zeros4):
    mesh = plsc.VectorSubcoreMesh(core_axis_name="c", subcore_axis_name="s")
    f32 = jnp.float32
    kern = functools.partial(
        pl.kernel,
        mesh=mesh,
        compiler_params=pltpu.CompilerParams(needs_layout_passes=False),
        out_type=jax.ShapeDtypeStruct((NP * 4,), f32),
        scratch_types=[
            pltpu.VMEM((GROUP * SLOTS,), jnp.int32),
            pltpu.VMEM((GROUP * CPAD,), jnp.int32),
            pltpu.VMEM((64,), jnp.int32),
            pltpu.VMEM((NP,), f32),
            pltpu.VMEM((NP,), f32),
            pltpu.VMEM((NP,), f32),
            pltpu.VMEM((NP,), f32),
            pltpu.VMEM((NP,), f32),
            pltpu.VMEM((CANDCAP,), jnp.int32),
            pltpu.VMEM((CANDCAP,), jnp.int32),
            pltpu.VMEM((CANDCAP,), f32),
            pltpu.VMEM((NP * 4,), f32),
            pltpu.VMEM((TPR * 4,), f32),
            pltpu.VMEM((16,), jnp.int32),
            pltpu.VMEM((NTILE * 16,), jnp.int32),
            pltpu.VMEM_SHARED((NTILE * 16,), jnp.int32),
            pltpu.VMEM_SHARED((NTILE * NP * 4,), f32),
        ],
    )
    return kern(_sc_body)(jtab, cnt, xmin, ymin, xmax, ymax, scores_p, zeros4)


def _passc_body(b_ref, d_ref, o_ref):
    o_ref[...] = b_ref[...] + d_ref[...]


def _passc(boxes_p, delta):
    return pl.pallas_call(
        _passc_body,
        out_shape=jax.ShapeDtypeStruct((NP, 4), jnp.float32),
    )(boxes_p, delta)


def _pad_cols(boxes):
    pad = NP - N
    xmin = jnp.pad(boxes[:, 0], (0, pad), constant_values=3.0e30)
    ymin = jnp.pad(boxes[:, 1], (0, pad), constant_values=3.0e30)
    xmax = jnp.pad(boxes[:, 2], (0, pad), constant_values=-3.0e30)
    ymax = jnp.pad(boxes[:, 3], (0, pad), constant_values=-3.0e30)
    return xmin, ymin, xmax, ymax


def kernel(boxes, scores):
    xmin, ymin, xmax, ymax = _pad_cols(boxes)
    jtab, cnt = _passa(xmin, ymin, xmax, ymax)
    scores_p = jnp.pad(scores, (0, NP - N))
    zeros4 = jnp.zeros((NP * 4,), jnp.float32)
    delta = _passb(jtab.reshape(NP * SLOTS), cnt.reshape(NP * CPAD),
                   xmin, ymin, xmax, ymax, scores_p, zeros4)
    boxes_p = jnp.pad(boxes, ((0, NP - N), (0, 0)))
    out = _passc(boxes_p, delta.reshape(NP, 4))
    return out[:N]
